# baseline (speedup 1.0000x reference)
import functools

import jax
import jax.numpy as jnp
from jax import lax
from jax.experimental import pallas as pl
from jax.experimental.pallas import tpu as pltpu
from jax.experimental.pallas import tpu_sc as plsc

STD = 1.0

CH = 192
ROWS = 512
WIDTH = 512
B = CH * ROWS
NC, NS = 2, 16
NW = NC * NS
BPW = B // NW
CHUNK = 96
NCHUNK = BPW // CHUNK
NBUF = 2
NGROUP = NCHUNK // NBUF


def _gather_rows(table, idx):
    mesh = plsc.VectorSubcoreMesh(core_axis_name="c", subcore_axis_name="s")

    @functools.partial(
        pl.kernel,
        out_type=jax.ShapeDtypeStruct((B, WIDTH), jnp.float32),
        mesh=mesh,
        scratch_types=[
            pltpu.VMEM((NCHUNK, CHUNK), jnp.int32),
            [pltpu.VMEM((CHUNK, WIDTH), jnp.float32) for _ in range(NBUF)],
            [pltpu.SemaphoreType.DMA for _ in range(NBUF)],
            [pltpu.SemaphoreType.DMA for _ in range(NBUF)],
        ],
    )
    def k(table_hbm, idx_hbm, out_hbm, idx_v, bufs, gsems, ssems):
        wid = lax.axis_index("s") * NC + lax.axis_index("c")
        base = wid * BPW
        pltpu.sync_copy(idx_hbm.at[wid], idx_v)

        def start_gather(j, b):
            pltpu.async_copy(table_hbm.at[idx_v.at[j]], bufs[b], gsems[b])

        def wait_gather(b):
            pltpu.make_async_copy(
                table_hbm.at[idx_v.at[0]], bufs[b], gsems[b]
            ).wait()

        def out_slab(j):
            return out_hbm.at[pl.ds(base + j * CHUNK, CHUNK)]

        def start_scatter(j, b):
            pltpu.async_copy(bufs[b], out_slab(j), ssems[b])

        def wait_scatter(b):
            pltpu.make_async_copy(bufs[b], out_slab(0), ssems[b]).wait()

        for b in range(NBUF):
            start_gather(b, b)

        def body(g, carry):
            j0 = g * NBUF
            for b in range(NBUF):
                wait_gather(b)
                start_scatter(j0 + b, b)
            for b in range(NBUF):
                wait_scatter(b)
                start_gather(j0 + NBUF + b, b)
            return carry

        lax.fori_loop(0, NGROUP - 1, body, 0)

        j0 = (NGROUP - 1) * NBUF
        for b in range(NBUF):
            wait_gather(b)
            start_scatter(j0 + b, b)
        for b in range(NBUF):
            wait_scatter(b)

    return k(table, idx)


def kernel(img):
    rows = img.shape[1]
    src = jnp.arange(0, rows)
    noise = jax.random.normal(jax.random.key(42), (rows,), dtype=jnp.float32)
    dst = jnp.clip(
        jnp.round(noise * STD + src.astype(jnp.float32)), 0, rows - 1
    ).astype(jnp.int32)

    flat_idx = (jnp.arange(CH, dtype=jnp.int32)[:, None] * ROWS + dst[None, :])
    flat_idx = flat_idx.reshape(NW, NCHUNK, CHUNK)

    table = img.reshape(B, WIDTH)
    out = _gather_rows(table, flat_idx)
    return out.reshape(CH, ROWS, WIDTH)


# speedup vs baseline: 1.0251x; 1.0251x over previous
import functools

import jax
import jax.numpy as jnp
from jax import lax
from jax.experimental import pallas as pl
from jax.experimental.pallas import tpu as pltpu
from jax.experimental.pallas import tpu_sc as plsc

STD = 1.0

CH = 192
ROWS = 512
WIDTH = 512
B = CH * ROWS
NC, NS = 2, 16
NW = NC * NS
BPW = B // NW
CHUNK = 24
NCHUNK = BPW // CHUNK
NBUF = 8
NGROUP = NCHUNK // NBUF


def _gather_rows(table, idx):
    mesh = plsc.VectorSubcoreMesh(core_axis_name="c", subcore_axis_name="s")

    @functools.partial(
        pl.kernel,
        out_type=jax.ShapeDtypeStruct((B, WIDTH), jnp.float32),
        mesh=mesh,
        scratch_types=[
            pltpu.VMEM((NCHUNK, CHUNK), jnp.int32),
            [pltpu.VMEM((CHUNK, WIDTH), jnp.float32) for _ in range(NBUF)],
            [pltpu.SemaphoreType.DMA for _ in range(NBUF)],
            [pltpu.SemaphoreType.DMA for _ in range(NBUF)],
        ],
    )
    def k(table_hbm, idx_hbm, out_hbm, idx_v, bufs, gsems, ssems):
        wid = lax.axis_index("s") * NC + lax.axis_index("c")
        base = wid * BPW
        pltpu.sync_copy(idx_hbm.at[wid], idx_v)

        def start_gather(j, b):
            pltpu.async_copy(table_hbm.at[idx_v.at[j]], bufs[b], gsems[b])

        def wait_gather(b):
            pltpu.make_async_copy(
                table_hbm.at[idx_v.at[0]], bufs[b], gsems[b]
            ).wait()

        def out_slab(j):
            return out_hbm.at[pl.ds(base + j * CHUNK, CHUNK)]

        def start_scatter(j, b):
            pltpu.async_copy(bufs[b], out_slab(j), ssems[b])

        def wait_scatter(b):
            pltpu.make_async_copy(bufs[b], out_slab(0), ssems[b]).wait()

        for b in range(NBUF):
            start_gather(b, b)

        def body(g, carry):
            j0 = g * NBUF
            for b in range(NBUF):
                wait_gather(b)
                start_scatter(j0 + b, b)
            for b in range(NBUF):
                wait_scatter(b)
                start_gather(j0 + NBUF + b, b)
            return carry

        lax.fori_loop(0, NGROUP - 1, body, 0)

        j0 = (NGROUP - 1) * NBUF
        for b in range(NBUF):
            wait_gather(b)
            start_scatter(j0 + b, b)
        for b in range(NBUF):
            wait_scatter(b)

    return k(table, idx)


def kernel(img):
    rows = img.shape[1]
    src = jnp.arange(0, rows)
    noise = jax.random.normal(jax.random.key(42), (rows,), dtype=jnp.float32)
    dst = jnp.clip(
        jnp.round(noise * STD + src.astype(jnp.float32)), 0, rows - 1
    ).astype(jnp.int32)

    flat_idx = (jnp.arange(CH, dtype=jnp.int32)[:, None] * ROWS + dst[None, :])
    flat_idx = flat_idx.reshape(NW, NCHUNK, CHUNK)

    table = img.reshape(B, WIDTH)
    out = _gather_rows(table, flat_idx)
    return out.reshape(CH, ROWS, WIDTH)
